# baseline (speedup 1.0000x reference)
import functools

import jax
import jax.numpy as jnp
from jax import lax
from jax.experimental import pallas as pl
from jax.experimental.pallas import tpu as pltpu
from jax.experimental.pallas import tpu_sc as plsc

_BUCKETS = 100
_DEMB = 64
_PAD = 79
_N = 4096 * 200
_S = _N * 2
_LANES_TC = 256
_ROWS_TC = _S // _LANES_TC
_BLK_TC = 256

_NW = 32
_PER_W = _S // _NW
_CHUNK = 512
_CROWS = _CHUNK // 2
_SUPER = 20
_SROWS = _SUPER * _CHUNK // _LANES_TC
_NSUPER = _PER_W // (_SUPER * _CHUNK)


def _tc_weights_body(x_ref, w1_ref, b1_ref, w2_ref, b2_ref, w_ref, i_ref):
    xv = x_ref[...]
    xn = pltpu.roll(xv, _LANES_TC - 1, 1)
    d = jnp.full(xv.shape, b2_ref[1, 0] - b2_ref[0, 0], jnp.float32)
    for o in range(32):
        h = xv * w1_ref[o, 0] + xn * w1_ref[o, 1] + b1_ref[o, 0]
        h = jnp.maximum(h, h * 0.01)
        d = d + (w2_ref[1, o] - w2_ref[0, o]) * h
    we = 1.0 / (1.0 + jnp.exp(d))
    lane = lax.broadcasted_iota(jnp.int32, xv.shape, 1)
    even = (lane % 2) == 0
    w_ref[...] = jnp.where(even, we, 1.0 - pltpu.roll(we, 1, 1))
    idx = jnp.floor(xv * jnp.float32(_BUCKETS)).astype(jnp.int32)
    i_ref[...] = jnp.clip(idx, 0, _BUCKETS - 1)


def _tc_weights(xr, W1, b1, W2, b2):
    grid = (_ROWS_TC // _BLK_TC,)
    blk = pl.BlockSpec((_BLK_TC, _LANES_TC), lambda i: (i, 0))
    rep2 = lambda shape: pl.BlockSpec(shape, lambda i: (0, 0))
    return pl.pallas_call(
        _tc_weights_body,
        grid=grid,
        in_specs=[blk, rep2((32, 2)), rep2((32, 1)), rep2((2, 32)), rep2((2, 1))],
        out_specs=[blk, blk],
        out_shape=[
            jax.ShapeDtypeStruct((_ROWS_TC, _LANES_TC), jnp.float32),
            jax.ShapeDtypeStruct((_ROWS_TC, _LANES_TC), jnp.int32),
        ],
    )(xr, W1, b1.reshape(32, 1), W2, b2.reshape(2, 1))


def _sc_body(idx_hbm, w_hbm, out_hbm,
             tab2_v, tab_v, idx_v, w_v, out_v, sem0, sem1):
    wid = lax.axis_index("c") * 16 + lax.axis_index("s")
    base_row = wid * (_PER_W // _LANES_TC)
    out_row_base = wid * (_PER_W // 2)
    pltpu.sync_copy(w_hbm.at[pl.ds(_ROWS_TC, 32), :], tab2_v)

    def unpack_row(r, c2):
        for k in range(16):
            tab_v[pl.ds(r * 256 + k * 16, 16)] = tab2_v[r, pl.ds(k * 16, 16)]
        return c2

    lax.fori_loop(0, 32, unpack_row, 0)
    iota = lax.broadcasted_iota(jnp.int32, (16,), 0)
    kcol = [(iota & 1) * _DEMB + ((iota + c) & (_DEMB - 1))
            for c in range(_DEMB)]
    rowi = iota // 2
    tabw = _BUCKETS * _PAD

    def fill(local_sub, buf):
        brow = buf * _CROWS

        def group(g, c2):
            s0 = local_sub + g * 16
            gr = s0 // _LANES_TC
            gl = s0 % _LANES_TC
            iv = plsc.bitcast(idx_v[gr, pl.ds(gl, 16)], jnp.int32)
            wv = w_v[gr, pl.ds(gl, 16)]
            a79 = iv * _PAD + iota
            rowv = brow + g * 8 + rowi
            for cw in range(0, _DEMB, 8):
                tvs = [
                    plsc.load_gather(tab_v.at[pl.ds(cw, tabw - cw)], [a79 + j])
                    for j in range(8)
                ]
                rs = [tv * wv for tv in tvs]
                for j in range(8):
                    plsc.store_scatter(out_v, [rowv, kcol[cw + j]], rs[j])
            return c2

        lax.fori_loop(0, _CHUNK // 16, group, 0)

    def superblock(sp, carry):
        srow = base_row + sp * _SROWS
        pltpu.sync_copy(idx_hbm.at[pl.ds(srow, _SROWS), :], idx_v)
        pltpu.sync_copy(w_hbm.at[pl.ds(srow, _SROWS), :], w_v)

        def pair(pj, c2):
            for b in range(2):
                ci = pj * 2 + b
                gci = sp * _SUPER + ci

                @pl.when(gci >= 2)
                def _wait():
                    sem = sem0 if b == 0 else sem1
                    pltpu.make_async_copy(
                        out_v.at[pl.ds(b * _CROWS, _CROWS), :],
                        out_hbm.at[pl.ds(0, _CROWS), :], sem).wait()

                fill(ci * _CHUNK, b)
                dst_row = out_row_base + (sp * _SUPER + ci) * _CROWS
                pltpu.async_copy(
                    out_v.at[pl.ds(b * _CROWS, _CROWS), :],
                    out_hbm.at[pl.ds(dst_row, _CROWS), :],
                    sem0 if b == 0 else sem1)
            return c2

        lax.fori_loop(0, _SUPER // 2, pair, 0)
        return carry

    lax.fori_loop(0, _NSUPER, superblock, 0)
    for b in range(2):
        pltpu.make_async_copy(
            out_v.at[pl.ds(b * _CROWS, _CROWS), :],
            out_hbm.at[pl.ds(0, _CROWS), :],
            sem0 if b == 0 else sem1).wait()


def _sc_gather(idx2, w2):
    mesh = plsc.VectorSubcoreMesh(core_axis_name="c", subcore_axis_name="s")
    k = functools.partial(
        pl.kernel,
        mesh=mesh,
        compiler_params=pltpu.CompilerParams(
            needs_layout_passes=False, use_tc_tiling_on_sc=True),
        out_type=jax.ShapeDtypeStruct((_N, 128), jnp.float32),
        scratch_types=[
            pltpu.VMEM((32, 256), jnp.float32),
            pltpu.VMEM((64 * 128,), jnp.float32),
            pltpu.VMEM((_SROWS, _LANES_TC), jnp.float32),
            pltpu.VMEM((_SROWS, _LANES_TC), jnp.float32),
            pltpu.VMEM((2 * _CROWS, 128), jnp.float32),
            pltpu.SemaphoreType.DMA,
            pltpu.SemaphoreType.DMA,
        ],
    )(_sc_body)
    return k(idx2, w2)


def kernel(x, emb_table, W1, b1, W2, b2):
    xr = x.reshape(_ROWS_TC, _LANES_TC)
    w, i = _tc_weights(xr, W1, b1, W2, b2)
    tab79 = jnp.concatenate([emb_table, emb_table[:, : _PAD - _DEMB]], axis=1)
    tab2 = jnp.pad(tab79.reshape(_BUCKETS * _PAD), (0, 64 * 128 - _BUCKETS * _PAD))
    w_plus = jnp.concatenate([w, tab2.reshape(32, _LANES_TC)], axis=0)
    out = _sc_gather(lax.bitcast_convert_type(i, jnp.float32), w_plus)
    return out.reshape(4096, 200, 128)


# speedup vs baseline: 2.8532x; 2.8532x over previous
import functools

import jax
import jax.numpy as jnp
from jax import lax
from jax.experimental import pallas as pl
from jax.experimental.pallas import tpu as pltpu
from jax.experimental.pallas import tpu_sc as plsc

_BUCKETS = 100
_DEMB = 64
_PAD = 79
_L = 200
_NB = 4096
_NW = 32
_LBLK_TC = 8
_STAGES = [(i * 24, 24) for i in range(8)] + [(192, 8)]


def _tc_weights_body(x_ref, w1_ref, b1_ref, w2_ref, b2_ref, w_ref, i_ref):
    xv = x_ref[...]
    xn = pltpu.roll(xv, _DEMB - 1, 1)
    d = jnp.full(xv.shape, b2_ref[1, 0] - b2_ref[0, 0], jnp.float32)
    for o in range(32):
        h = xv * w1_ref[o, 0] + xn * w1_ref[o, 1] + b1_ref[o, 0]
        h = jnp.maximum(h, h * 0.01)
        d = d + (w2_ref[1, o] - w2_ref[0, o]) * h
    we = 1.0 / (1.0 + jnp.exp(d))
    m = lax.broadcasted_iota(jnp.int32, xv.shape, 1)
    even = (m % 2) == 0
    w_ref[...] = jnp.where(even, we, 1.0 - pltpu.roll(we, 1, 1))
    idx = jnp.floor(xv * jnp.float32(_BUCKETS)).astype(jnp.int32)
    i_ref[...] = jnp.clip(idx, 0, _BUCKETS - 1)


def _tc_weights(xp, W1, b1, W2, b2):
    grid = (_L // _LBLK_TC,)
    blk = pl.BlockSpec((_LBLK_TC, _DEMB, 128), lambda i: (i, 0, 0))
    rep2 = lambda shape: pl.BlockSpec(shape, lambda i: (0, 0))
    return pl.pallas_call(
        _tc_weights_body,
        grid=grid,
        in_specs=[blk, rep2((32, 2)), rep2((32, 1)), rep2((2, 32)), rep2((2, 1))],
        out_specs=[blk, blk],
        out_shape=[
            jax.ShapeDtypeStruct((_L, _DEMB, 128), jnp.float32),
            jax.ShapeDtypeStruct((_L, _DEMB, 128), jnp.int32),
        ],
    )(xp, W1, b1.reshape(32, 1), W2, b2.reshape(2, 1))


def _sc_body(idx_hbm, w_hbm, out_hbm,
             tab2_v, tab_v, idx_v, w_v, out_v, sem0, sem1):
    wid = lax.axis_index("c") * 16 + lax.axis_index("s")
    m8 = (wid // 4) * 8
    msub = (wid % 4) * 2
    pltpu.sync_copy(w_hbm.at[pl.ds(_L, 1), :, :], tab2_v)

    def unpack_row(r, c2):
        for k in range(8):
            tab_v[pl.ds(r * 128 + k * 16, 16)] = tab2_v[0, r, pl.ds(k * 16, 16)]
        return c2

    lax.fori_loop(0, _DEMB, unpack_row, 0)
    iota = lax.broadcasted_iota(jnp.int32, (16,), 0)
    rot = [(iota + c) & (_DEMB - 1) for c in range(_DEMB)]
    bj0 = [iota, iota + 16]
    tabw = _BUCKETS * _PAD

    def fill(bjblk, lloc0, nl, buf):
        def group(lf, c2):
            lloc = lloc0 + lf // 2
            f = lf % 2
            iv = plsc.bitcast(idx_v[lloc, msub + f, pl.ds(bjblk * 16, 16)],
                              jnp.int32)
            wv = w_v[lloc, msub + f, pl.ds(bjblk * 16, 16)]
            a79 = iv * _PAD + iota
            lrv = jnp.full((16,), lf // 2, jnp.int32)
            f64 = f * _DEMB
            for cw in range(0, _DEMB, 8):
                tvs = [
                    plsc.load_gather(tab_v.at[pl.ds(cw, tabw - cw)], [a79 + j])
                    for j in range(8)
                ]
                rs = [tv * wv for tv in tvs]
                for j in range(8):
                    plsc.store_scatter(
                        out_v, [bj0[buf], lrv, rot[cw + j] + f64], rs[j])
            return c2

        lax.fori_loop(0, 2 * nl, group, 0)

    nl = 8

    def do_stage(l0, lsz, base):
        pltpu.sync_copy(idx_hbm.at[pl.ds(l0, lsz), pl.ds(m8, 8), :],
                        idx_v.at[pl.ds(0, lsz), :, :])
        pltpu.sync_copy(w_hbm.at[pl.ds(l0, lsz), pl.ds(m8, 8), :],
                        w_v.at[pl.ds(0, lsz), :, :])
        nlb = lsz // nl
        nchunk = nlb * 8

        def pair(pj, c2):
            for b in range(2):
                ci = pj * 2 + b
                lblk = ci % nlb
                bjblk = ci // nlb

                @pl.when(base + ci >= 2)
                def _wait():
                    sem = sem0 if b == 0 else sem1
                    pltpu.make_async_copy(
                        out_v.at[pl.ds(b * 16, 16), :, :],
                        out_hbm.at[pl.ds(0, 16), pl.ds(0, nl), :],
                        sem).wait()

                fill(bjblk, lblk * nl, nl, b)
                l_off = pl.multiple_of(l0 + lblk * nl, 8)
                pltpu.async_copy(
                    out_v.at[pl.ds(b * 16, 16), :, :],
                    out_hbm.at[pl.ds(wid * 128 + bjblk * 16, 16),
                               pl.ds(l_off, nl), :],
                    sem0 if b == 0 else sem1)
            return c2

        lax.fori_loop(0, nchunk // 2, pair, 0)

    lax.fori_loop(0, 8, lambda si, c: (do_stage(si * 24, 24, si * 192), c)[1], 0)
    do_stage(192, 8, 8 * 192)
    for b in range(2):
        pltpu.make_async_copy(
            out_v.at[pl.ds(b * 16, 16), :, :],
            out_hbm.at[pl.ds(0, 16), pl.ds(0, nl), :],
            sem0 if b == 0 else sem1).wait()


def _sc_gather(idxp, wp):
    mesh = plsc.VectorSubcoreMesh(core_axis_name="c", subcore_axis_name="s")
    k = functools.partial(
        pl.kernel,
        mesh=mesh,
        compiler_params=pltpu.CompilerParams(
            needs_layout_passes=False, use_tc_tiling_on_sc=True),
        out_type=jax.ShapeDtypeStruct((_NB, _L, 128), jnp.float32),
        scratch_types=[
            pltpu.VMEM((1, _DEMB, 128), jnp.float32),
            pltpu.VMEM((_DEMB * 128,), jnp.float32),
            pltpu.VMEM((24, 8, 128), jnp.float32),
            pltpu.VMEM((24, 8, 128), jnp.float32),
            pltpu.VMEM((32, 8, 128), jnp.float32),
            pltpu.SemaphoreType.DMA,
            pltpu.SemaphoreType.DMA,
        ],
    )(_sc_body)
    return k(idxp, wp)


def kernel(x, emb_table, W1, b1, W2, b2):
    xp = (x.reshape(32, 128, _L, 2).transpose(2, 0, 3, 1)
          .reshape(_L, _DEMB, 128))
    w, i = _tc_weights(xp, W1, b1, W2, b2)
    tab79 = jnp.concatenate([emb_table, emb_table[:, : _PAD - _DEMB]], axis=1)
    tab2 = jnp.pad(tab79.reshape(_BUCKETS * _PAD),
                   (0, _DEMB * 128 - _BUCKETS * _PAD))
    w_plus = jnp.concatenate([w, tab2.reshape(1, _DEMB, 128)], axis=0)
    return _sc_gather(lax.bitcast_convert_type(i, jnp.float32), w_plus)
